# baseline (speedup 1.0000x reference)
import jax
import jax.numpy as jnp
from jax import lax
from jax.experimental import pallas as pl
from jax.experimental.pallas import tpu as pltpu
from jax.experimental.pallas import tpu_sc as plsc

NC = 2
NS = 16
NW = NC * NS

NF = 16384
K = 26
D = 32
FPW = NF // NW
IPW = FPW * K
F = 4
CI = F * K
NCHUNK = FPW // F


def _body(feat_hbm, table_hbm, out_hbm, idx_v, buf_v, stage_v, g0, g1, w0, w1):
    wid = lax.axis_index("s") * NC + lax.axis_index("c")
    base = wid * IPW
    f_base = wid * FPW

    pltpu.sync_copy(feat_hbm.at[pl.ds(base, IPW)], idx_v)

    gsem = (g0, g1)
    wsem = (w0, w1)

    def _gather_args(c, b):
        return (table_hbm.at[idx_v.at[pl.ds(c * CI, CI)]], buf_v.at[b],
                gsem[b])

    def _write_args(c, b, fl):
        return (stage_v.at[b, fl], out_hbm.at[f_base + c * F + fl], wsem[b])

    def fill(b):
        for fl in range(F):
            for j in range(K):
                r = fl * K + j
                stage_v[b, fl, j, pl.ds(0, 16)] = buf_v[b, r, pl.ds(0, 16)]
                stage_v[b, fl, j, pl.ds(16, 16)] = buf_v[b, r, pl.ds(16, 16)]

    pltpu.async_copy(*_gather_args(0, 0))
    pltpu.async_copy(*_gather_args(1, 1))

    def step(s, carry):
        for p in (0, 1):
            c = 2 * s + p
            pltpu.make_async_copy(*_gather_args(c, p)).wait()

            @pl.when(c >= 2)
            def _reuse_stage():
                for fl in range(F):
                    pltpu.make_async_copy(*_write_args(c - 2, p, fl)).wait()

            fill(p)
            for fl in range(F):
                pltpu.async_copy(*_write_args(c, p, fl))

            @pl.when(c + 2 < NCHUNK)
            def _next_gather():
                pltpu.async_copy(*_gather_args(c + 2, p))
        return carry

    lax.fori_loop(0, NCHUNK // 2, step, 0)
    for fl in range(F):
        pltpu.make_async_copy(*_write_args(NCHUNK - 2, 0, fl)).wait()
        pltpu.make_async_copy(*_write_args(NCHUNK - 1, 1, fl)).wait()


@jax.jit
def _lookup(feats, t128):
    mesh = plsc.VectorSubcoreMesh(core_axis_name="c", subcore_axis_name="s")
    return pl.kernel(
        _body,
        out_type=jax.ShapeDtypeStruct((NF, K, D), jnp.float32),
        mesh=mesh,
        compiler_params=pltpu.CompilerParams(use_tc_tiling_on_sc=True),
        scratch_types=[
            pltpu.VMEM((IPW,), jnp.int32),
            pltpu.VMEM((2, CI, 128), jnp.float32),
            pltpu.VMEM((2, F, K, D), jnp.float32),
            pltpu.SemaphoreType.DMA,
            pltpu.SemaphoreType.DMA,
            pltpu.SemaphoreType.DMA,
            pltpu.SemaphoreType.DMA,
        ],
    )(feats, t128)


def kernel(features, table):
    feats = features.reshape(-1).astype(jnp.int32)
    t128 = jnp.pad(table.at[0].set(0.0), ((0, 7), (0, 96)))
    return _lookup(feats, t128)


# speedup vs baseline: 1.1838x; 1.1838x over previous
import jax
import jax.numpy as jnp
from jax import lax
from jax.experimental import pallas as pl
from jax.experimental.pallas import tpu as pltpu
from jax.experimental.pallas import tpu_sc as plsc

NC = 2
NS = 16
NW = NC * NS

NF = 16384
K = 26
D = 32
B = NF * K
SLABS = 4
BS = B // SLABS
BPW = BS // NW
C = 832
NCHUNK = BPW // C


def _make_body(slab):
    def _body(feat_hbm, table_hbm, out_hbm, idx_v, buf_v, g0, g1, w0, w1):
        wid = lax.axis_index("s") * NC + lax.axis_index("c")
        gbase = slab * BS + wid * BPW
        base = wid * BPW

        pltpu.sync_copy(feat_hbm.at[pl.ds(gbase, BPW)], idx_v)

        gsem = (g0, g1)
        wsem = (w0, w1)

        def _gather_args(c, b):
            return (table_hbm.at[idx_v.at[pl.ds(c * C, C)]], buf_v.at[b],
                    gsem[b])

        def _write_args(c, b):
            return (buf_v.at[b], out_hbm.at[pl.ds(base + c * C, C)], wsem[b])

        pltpu.async_copy(*_gather_args(0, 0))
        for c in range(NCHUNK):
            b = c & 1
            if c + 1 < NCHUNK:
                if c >= 1:
                    pltpu.make_async_copy(*_write_args(c - 1, 1 - b)).wait()
                pltpu.async_copy(*_gather_args(c + 1, 1 - b))
            pltpu.make_async_copy(*_gather_args(c, b)).wait()
            pltpu.async_copy(*_write_args(c, b))
        pltpu.make_async_copy(*_write_args(NCHUNK - 2, (NCHUNK - 2) & 1)).wait()
        pltpu.make_async_copy(*_write_args(NCHUNK - 1, (NCHUNK - 1) & 1)).wait()

    return _body


@jax.jit
def _lookup(feats, table):
    mesh = plsc.VectorSubcoreMesh(core_axis_name="c", subcore_axis_name="s")
    outs = []
    for s in range(SLABS):
        o = pl.kernel(
            _make_body(s),
            out_type=jax.ShapeDtypeStruct((BS, D), jnp.float32),
            mesh=mesh,
            compiler_params=pltpu.CompilerParams(use_tc_tiling_on_sc=False),
            scratch_types=[
                pltpu.VMEM((BPW,), jnp.int32),
                pltpu.VMEM((2, C, D), jnp.float32),
                pltpu.SemaphoreType.DMA,
                pltpu.SemaphoreType.DMA,
                pltpu.SemaphoreType.DMA,
                pltpu.SemaphoreType.DMA,
            ],
        )(feats, table)
        outs.append(o.reshape(NF // SLABS, K, D))
    return jnp.concatenate(outs, axis=0)


def kernel(features, table):
    feats = features.reshape(-1).astype(jnp.int32)
    t = table.at[0].set(0.0)
    return _lookup(feats, t)
